# baseline (speedup 1.0000x reference)
import functools

import jax
import jax.numpy as jnp
from jax import lax
from jax.experimental import pallas as pl
from jax.experimental.pallas import tpu as pltpu
from jax.experimental.pallas import tpu_sc as plsc

LANE = 16
BLK = 128


@functools.partial(jax.jit, static_argnames=("n_rows", "n_cols", "n_workers"))
def _embed_sc(xt, table_p, n_rows, n_cols, n_workers):
    d = 64
    assert table_p.shape[1] == BLK
    assert n_rows == n_workers * BLK
    assert n_cols % 2 == 0 and n_cols >= 8

    mesh = plsc.VectorSubcoreMesh(core_axis_name="c", subcore_axis_name="s")

    @functools.partial(
        pl.kernel,
        out_type=jax.ShapeDtypeStruct((n_cols, d, n_rows), jnp.float32),
        mesh=mesh,
        scratch_types=[
            pltpu.VMEM((n_cols, BLK), jnp.int32),
            pltpu.VMEM((2, BLK, BLK), jnp.float32),
            pltpu.VMEM((2, d, BLK), jnp.float32),
        ]
        + [pltpu.SemaphoreType.DMA] * 4,
        compiler_params=pltpu.CompilerParams(
            use_tc_tiling_on_sc=True, needs_layout_passes=False
        ),
    )
    def k(xt_hbm, tab_hbm, out_hbm, idx_v, rows_v, bt_v, *sems):
        gsem = sems[:2]
        wsem = sems[2:]
        wid = lax.axis_index("s") * 2 + lax.axis_index("c")
        lane0 = wid * BLK

        pltpu.sync_copy(xt_hbm.at[:, pl.ds(lane0, BLK)], idx_v)

        row_ids = [lax.iota(jnp.int32, LANE) + g * LANE for g in range(BLK // LANE)]

        def start_gather(j, b):
            pltpu.make_async_copy(
                tab_hbm.at[idx_v.at[j]], rows_v.at[b], gsem[b]
            ).start()

        def wait_gather(b):
            pltpu.make_async_copy(
                tab_hbm.at[idx_v.at[0]], rows_v.at[b], gsem[b]
            ).wait()

        def start_write(j, b):
            pltpu.make_async_copy(
                bt_v.at[b], out_hbm.at[j, :, pl.ds(lane0, BLK)], wsem[b]
            ).start()

        def wait_write(b):
            pltpu.make_async_copy(
                bt_v.at[b], out_hbm.at[0, :, pl.ds(lane0, BLK)], wsem[b]
            ).wait()

        def transpose(b):
            for dd in range(d):
                col = jnp.full((LANE,), dd, jnp.int32)
                for g in range(BLK // LANE):
                    vals = plsc.load_gather(rows_v.at[b], [row_ids[g], col])
                    bt_v[b, dd, pl.ds(g * LANE, LANE)] = vals

        start_gather(0, 0)
        start_gather(1, 1)
        for b in range(2):
            wait_gather(b)
            transpose(b)
            start_write(b, b)
            start_gather(b + 2, b)

        def pair(g, carry):
            for b in range(2):
                j = 2 * g + b
                wait_gather(b)
                wait_write(b)
                transpose(b)
                start_write(j, b)
                start_gather(j + 2, b)
            return carry

        lax.fori_loop(1, n_cols // 2 - 1, pair, 0)

        for b in range(2):
            j = n_cols - 2 + b
            wait_gather(b)
            wait_write(b)
            transpose(b)
            start_write(j, b)
        for b in range(2):
            wait_write(b)

    return k(xt, table_p)


def kernel(x, table):
    n_rows, n_cols = x.shape
    d = table.shape[1]
    xt = jnp.swapaxes(x, 0, 1).astype(jnp.int32)
    table_p = jnp.pad(table, ((0, 0), (0, BLK - d)))
    out = _embed_sc(xt, table_p, n_rows, n_cols, 32)
    return jnp.transpose(out, (2, 0, 1))


# speedup vs baseline: 2.0800x; 2.0800x over previous
import functools

import jax
import jax.numpy as jnp
from jax import lax
from jax.experimental import pallas as pl
from jax.experimental.pallas import tpu as pltpu
from jax.experimental.pallas import tpu_sc as plsc

LANE = 16
BLK = 128


@functools.partial(jax.jit, static_argnames=("n_rows", "n_cols", "n_workers"))
def _embed_sc(xt, table_p, n_rows, n_cols, n_workers):
    d = 64
    assert table_p.shape[1] == BLK
    assert n_rows == n_workers * BLK
    assert n_cols % 2 == 0 and n_cols >= 8

    mesh = plsc.VectorSubcoreMesh(core_axis_name="c", subcore_axis_name="s")

    @functools.partial(
        pl.kernel,
        out_type=jax.ShapeDtypeStruct((n_cols, d, n_rows), jnp.float32),
        mesh=mesh,
        scratch_types=[
            pltpu.VMEM((n_cols, BLK), jnp.int32),
            pltpu.VMEM((2, BLK, BLK), jnp.float32),
            pltpu.VMEM((2, d, BLK), jnp.float32),
        ]
        + [pltpu.SemaphoreType.DMA] * 4,
        compiler_params=pltpu.CompilerParams(
            use_tc_tiling_on_sc=True, needs_layout_passes=False
        ),
    )
    def k(xt_hbm, tab_hbm, out_hbm, idx_v, rows_v, bt_v, *sems):
        gsem = sems[:2]
        wsem = sems[2:]
        wid = lax.axis_index("s") * 2 + lax.axis_index("c")
        lane0 = wid * BLK

        pltpu.sync_copy(xt_hbm.at[:, pl.ds(lane0, BLK)], idx_v)

        row_ids = [lax.iota(jnp.int32, LANE) + g * LANE for g in range(BLK // LANE)]

        def start_gather(j, b):
            pltpu.make_async_copy(
                tab_hbm.at[idx_v.at[j]], rows_v.at[b], gsem[b]
            ).start()

        def wait_gather(b):
            pltpu.make_async_copy(
                tab_hbm.at[idx_v.at[0]], rows_v.at[b], gsem[b]
            ).wait()

        def start_write(j, b):
            pltpu.make_async_copy(
                bt_v.at[b], out_hbm.at[j, :, pl.ds(lane0, BLK)], wsem[b]
            ).start()

        def wait_write(b):
            pltpu.make_async_copy(
                bt_v.at[b], out_hbm.at[0, :, pl.ds(lane0, BLK)], wsem[b]
            ).wait()

        def transpose(b):
            @plsc.parallel_loop(0, d, unroll=4)
            def _(dd):
                col = jnp.full((LANE,), dd, jnp.int32)
                for g in range(BLK // LANE):
                    vals = plsc.load_gather(rows_v.at[b], [row_ids[g], col])
                    bt_v[b, dd, pl.ds(g * LANE, LANE)] = vals

        start_gather(0, 0)
        start_gather(1, 1)
        for b in range(2):
            wait_gather(b)
            transpose(b)
            start_write(b, b)
            start_gather(b + 2, b)

        def pair(g, carry):
            for b in range(2):
                j = 2 * g + b
                wait_gather(b)
                wait_write(b)
                transpose(b)
                start_write(j, b)
                start_gather(j + 2, b)
            return carry

        lax.fori_loop(1, n_cols // 2 - 1, pair, 0)

        for b in range(2):
            j = n_cols - 2 + b
            wait_gather(b)
            wait_write(b)
            transpose(b)
            start_write(j, b)
        for b in range(2):
            wait_write(b)

    return k(xt, table_p)


def kernel(x, table):
    n_rows, n_cols = x.shape
    d = table.shape[1]
    xt = jnp.swapaxes(x, 0, 1).astype(jnp.int32)
    table_p = jnp.pad(table, ((0, 0), (0, BLK - d)))
    out = _embed_sc(xt, table_p, n_rows, n_cols, 32)
    return jnp.transpose(out, (2, 0, 1))


# speedup vs baseline: 4.5279x; 2.1769x over previous
import functools

import jax
import jax.numpy as jnp
from jax import lax
from jax.experimental import pallas as pl
from jax.experimental.pallas import tpu as pltpu
from jax.experimental.pallas import tpu_sc as plsc

LANE = 16
BLK = 128


@functools.partial(jax.jit, static_argnames=("n_rows", "n_cols", "n_workers"))
def _embed_sc(xt, table_p, n_rows, n_cols, n_workers):
    d = 64
    assert table_p.shape[1] == BLK
    assert n_rows == n_workers * BLK
    assert n_cols % 2 == 0 and n_cols >= 8

    mesh = plsc.VectorSubcoreMesh(core_axis_name="c", subcore_axis_name="s")

    @functools.partial(
        pl.kernel,
        out_type=jax.ShapeDtypeStruct((n_cols, d, n_rows), jnp.float32),
        mesh=mesh,
        scratch_types=[
            pltpu.VMEM((n_cols, BLK), jnp.int32),
            pltpu.VMEM((2, BLK, BLK), jnp.float32),
            pltpu.VMEM((2, d, BLK), jnp.float32),
        ]
        + [pltpu.SemaphoreType.DMA] * 4,
        compiler_params=pltpu.CompilerParams(
            use_tc_tiling_on_sc=True, needs_layout_passes=False
        ),
    )
    def k(xt_hbm, tab_hbm, out_hbm, idx_v, rows_v, bt_v, *sems):
        gsem = sems[:2]
        wsem = sems[2:]
        wid = lax.axis_index("s") * 2 + lax.axis_index("c")
        lane0 = wid * BLK

        pltpu.sync_copy(xt_hbm.at[:, pl.ds(lane0, BLK)], idx_v)

        row_ids = [lax.iota(jnp.int32, LANE) + g * LANE for g in range(BLK // LANE)]

        def start_gather(j, b):
            pltpu.make_async_copy(
                tab_hbm.at[idx_v.at[j]], rows_v.at[b], gsem[b]
            ).start()

        def wait_gather(b):
            pltpu.make_async_copy(
                tab_hbm.at[idx_v.at[0]], rows_v.at[b], gsem[b]
            ).wait()

        def start_write(j, b):
            pltpu.make_async_copy(
                bt_v.at[b], out_hbm.at[j, :, pl.ds(lane0, BLK)], wsem[b]
            ).start()

        def wait_write(b):
            pltpu.make_async_copy(
                bt_v.at[b], out_hbm.at[0, :, pl.ds(lane0, BLK)], wsem[b]
            ).wait()

        lanes = lax.iota(jnp.int32, LANE)

        def transpose(b):
            @plsc.parallel_loop(0, LANE, unroll=2)
            def _(s):
                perm = (lanes + s) & (LANE - 1)
                for db in range(d // LANE):
                    cvec = perm + db * LANE
                    for g in range(BLK // LANE):
                        vals = plsc.load_gather(rows_v.at[b], [row_ids[g], cvec])
                        plsc.store_scatter(bt_v.at[b], [cvec, row_ids[g]], vals)

        start_gather(0, 0)
        start_gather(1, 1)
        for b in range(2):
            wait_gather(b)
            transpose(b)
            start_write(b, b)
            start_gather(b + 2, b)

        def pair(g, carry):
            for b in range(2):
                j = 2 * g + b
                wait_gather(b)
                wait_write(b)
                transpose(b)
                start_write(j, b)
                start_gather(j + 2, b)
            return carry

        lax.fori_loop(1, n_cols // 2 - 1, pair, 0)

        for b in range(2):
            j = n_cols - 2 + b
            wait_gather(b)
            wait_write(b)
            transpose(b)
            start_write(j, b)
        for b in range(2):
            wait_write(b)

    return k(xt, table_p)


def kernel(x, table):
    n_rows, n_cols = x.shape
    d = table.shape[1]
    xt = jnp.swapaxes(x, 0, 1).astype(jnp.int32)
    table_p = jnp.pad(table, ((0, 0), (0, BLK - d)))
    out = _embed_sc(xt, table_p, n_rows, n_cols, 32)
    return jnp.transpose(out, (2, 0, 1))


# speedup vs baseline: 4.7024x; 1.0386x over previous
import functools

import jax
import jax.numpy as jnp
from jax import lax
from jax.experimental import pallas as pl
from jax.experimental.pallas import tpu as pltpu
from jax.experimental.pallas import tpu_sc as plsc

LANE = 16
BLK = 128


@functools.partial(jax.jit, static_argnames=("n_rows", "n_cols", "n_workers"))
def _embed_sc(xt, table_p, n_rows, n_cols, n_workers):
    d = 64
    assert table_p.shape[1] == BLK
    assert n_rows == n_workers * BLK
    assert n_cols % 2 == 0 and n_cols >= 8

    mesh = plsc.VectorSubcoreMesh(core_axis_name="c", subcore_axis_name="s")

    @functools.partial(
        pl.kernel,
        out_type=jax.ShapeDtypeStruct((n_cols, d, n_rows), jnp.float32),
        mesh=mesh,
        scratch_types=[
            pltpu.VMEM((n_cols, BLK), jnp.int32),
            pltpu.VMEM((2, BLK, BLK), jnp.float32),
            pltpu.VMEM((2, d, BLK), jnp.float32),
        ]
        + [pltpu.SemaphoreType.DMA] * 4,
        compiler_params=pltpu.CompilerParams(
            use_tc_tiling_on_sc=True, needs_layout_passes=False
        ),
    )
    def k(xt_hbm, tab_hbm, out_hbm, idx_v, rows_v, bt_v, *sems):
        gsem = sems[:2]
        wsem = sems[2:]
        wid = lax.axis_index("s") * 2 + lax.axis_index("c")
        lane0 = wid * BLK

        pltpu.sync_copy(xt_hbm.at[:, pl.ds(lane0, BLK)], idx_v)

        row_ids = [lax.iota(jnp.int32, LANE) + g * LANE for g in range(BLK // LANE)]

        def start_gather(j, b):
            pltpu.make_async_copy(
                tab_hbm.at[idx_v.at[j]], rows_v.at[b], gsem[b]
            ).start()

        def wait_gather(b):
            pltpu.make_async_copy(
                tab_hbm.at[idx_v.at[0]], rows_v.at[b], gsem[b]
            ).wait()

        def start_write(j, b):
            pltpu.make_async_copy(
                bt_v.at[b], out_hbm.at[j, :, pl.ds(lane0, BLK)], wsem[b]
            ).start()

        def wait_write(b):
            pltpu.make_async_copy(
                bt_v.at[b], out_hbm.at[0, :, pl.ds(lane0, BLK)], wsem[b]
            ).wait()

        lanes = lax.iota(jnp.int32, LANE)

        def transpose(b):
            @plsc.parallel_loop(0, LANE, unroll=4)
            def _(s):
                perm = (lanes + s) & (LANE - 1)
                for db in range(d // LANE):
                    cvec = perm + db * LANE
                    for g in range(BLK // LANE):
                        vals = plsc.load_gather(rows_v.at[b], [row_ids[g], cvec])
                        plsc.store_scatter(bt_v.at[b], [cvec, row_ids[g]], vals)

        start_gather(0, 0)
        start_gather(1, 1)
        for b in range(2):
            wait_gather(b)
            transpose(b)
            start_write(b, b)
            start_gather(b + 2, b)

        def pair(g, carry):
            for b in range(2):
                j = 2 * g + b
                wait_gather(b)
                wait_write(b)
                transpose(b)
                start_write(j, b)
                start_gather(j + 2, b)
            return carry

        lax.fori_loop(1, n_cols // 2 - 1, pair, 0)

        for b in range(2):
            j = n_cols - 2 + b
            wait_gather(b)
            wait_write(b)
            transpose(b)
            start_write(j, b)
        for b in range(2):
            wait_write(b)

    return k(xt, table_p)


def kernel(x, table):
    n_rows, n_cols = x.shape
    d = table.shape[1]
    xt = jnp.swapaxes(x, 0, 1).astype(jnp.int32)
    table_p = jnp.pad(table, ((0, 0), (0, BLK - d)))
    out = _embed_sc(xt, table_p, n_rows, n_cols, 32)
    return jnp.transpose(out, (2, 0, 1))
